# baseline (speedup 1.0000x reference)
import functools

import jax
import jax.numpy as jnp
from jax import lax
from jax.experimental import pallas as pl
from jax.experimental.pallas import tpu as pltpu
from jax.experimental.pallas import tpu_sc as plsc

_WINDOW = 512
_D = 16384
_OUT_SLICES = (0, 127, 255, 383, 511)
_SRC_ROWS = tuple(s + 1 for s in _OUT_SLICES if s < _WINDOW - 1)
_NROWS = len(_OUT_SLICES)

_NS = 16
_CW = _D // _NS

_mesh = plsc.VectorSubcoreMesh(core_axis_name="c", subcore_axis_name="s",
                               num_cores=1)


@functools.partial(
    pl.kernel,
    mesh=_mesh,
    out_type=jax.ShapeDtypeStruct((_NROWS, _D), jnp.float32),
    scratch_types=[
        pltpu.VMEM((_NROWS, _CW), jnp.float32),
        pltpu.SemaphoreType.DMA,
        pltpu.SemaphoreType.DMA,
    ],
)
def _gather_rows(x_hbm, buf4_hbm, out_hbm, vbuf, sem_in, sem_out):
    t = lax.axis_index("s")
    base = t * _CW
    a = pltpu.async_copy(
        buf4_hbm.at[pl.ds(1, 3), 0, pl.ds(base, _CW)],
        vbuf.at[pl.ds(1, 3), :],
        sem_in)
    b = pltpu.async_copy(
        buf4_hbm.at[0, pl.ds(1, 1), pl.ds(base, _CW)],
        vbuf.at[pl.ds(0, 1), :],
        sem_in)
    c = pltpu.async_copy(
        x_hbm.at[pl.ds(0, 1), pl.ds(base, _CW)],
        vbuf.at[pl.ds(_NROWS - 1, 1), :],
        sem_in)
    a.wait()
    b.wait()
    c.wait()
    pltpu.async_copy(vbuf, out_hbm.at[:, pl.ds(base, _CW)], sem_out).wait()


def kernel(x, buffer):
    return _gather_rows(x.reshape(1, _D), buffer.reshape(4, _WINDOW // 4, _D))


# speedup vs baseline: 1.0633x; 1.0633x over previous
import functools

import jax
import jax.numpy as jnp
from jax import lax
from jax.experimental import pallas as pl
from jax.experimental.pallas import tpu as pltpu
from jax.experimental.pallas import tpu_sc as plsc

_WINDOW = 512
_D = 16384
_OUT_SLICES = (0, 127, 255, 383, 511)
_SRC_ROWS = tuple(s + 1 for s in _OUT_SLICES if s < _WINDOW - 1)
_NROWS = len(_OUT_SLICES)

_NS = 16
_CW = _D // _NS

_mesh = plsc.ScalarSubcoreMesh(axis_name="c", num_cores=1)


@functools.partial(
    pl.kernel,
    mesh=_mesh,
    out_type=jax.ShapeDtypeStruct((_NROWS, _D), jnp.float32),
    scratch_types=[
        pltpu.VMEM_SHARED((_NROWS, _D), jnp.float32),
        pltpu.SemaphoreType.DMA,
        pltpu.SemaphoreType.DMA,
    ],
)
def _gather_rows(x_hbm, buf4_hbm, out_hbm, vbuf, sem_in, sem_out):
    a = pltpu.async_copy(
        buf4_hbm.at[pl.ds(1, 3), 0, :],
        vbuf.at[pl.ds(1, 3), :],
        sem_in)
    b = pltpu.async_copy(
        buf4_hbm.at[0, pl.ds(1, 1), :],
        vbuf.at[pl.ds(0, 1), :],
        sem_in)
    c = pltpu.async_copy(
        x_hbm.at[pl.ds(0, 1), :],
        vbuf.at[pl.ds(_NROWS - 1, 1), :],
        sem_in)
    a.wait()
    b.wait()
    c.wait()
    pltpu.async_copy(vbuf, out_hbm, sem_out).wait()


def kernel(x, buffer):
    return _gather_rows(x.reshape(1, _D), buffer.reshape(4, _WINDOW // 4, _D))


# speedup vs baseline: 1.0684x; 1.0048x over previous
import functools

import jax
import jax.numpy as jnp
from jax.experimental import pallas as pl
from jax.experimental.pallas import tpu as pltpu
from jax.experimental.pallas import tpu_sc as plsc

_WINDOW = 512
_D = 16384
_OUT_SLICES = (0, 127, 255, 383, 511)
_SRC_ROWS = tuple(s + 1 for s in _OUT_SLICES if s < _WINDOW - 1)
_NROWS = len(_OUT_SLICES)

_mesh = plsc.ScalarSubcoreMesh(axis_name="c", num_cores=1)


@functools.partial(
    pl.kernel,
    mesh=_mesh,
    out_type=jax.ShapeDtypeStruct((_NROWS, _D), jnp.float32),
    scratch_types=[
        pltpu.VMEM_SHARED((_NROWS, _D), jnp.float32),
        pltpu.SemaphoreType.DMA,
        pltpu.SemaphoreType.DMA,
    ],
)
def _gather_rows(x_hbm, buf4_hbm, out_hbm, vbuf, sem_in, sem_out):
    a = pltpu.async_copy(
        buf4_hbm.at[pl.ds(1, 3), 0, :],
        vbuf.at[pl.ds(1, 3), :],
        sem_in)
    b = pltpu.async_copy(
        buf4_hbm.at[0, pl.ds(1, 1), :],
        vbuf.at[pl.ds(0, 1), :],
        sem_in)
    c = pltpu.async_copy(
        x_hbm.at[pl.ds(0, 1), :],
        vbuf.at[pl.ds(_NROWS - 1, 1), :],
        sem_in)
    a.wait()
    b.wait()
    c.wait()
    pltpu.async_copy(vbuf, out_hbm, sem_out).wait()


def kernel(x, buffer):
    return _gather_rows(x.reshape(1, _D), buffer.reshape(4, _WINDOW // 4, _D))
